# Initial kernel scaffold; baseline (speedup 1.0000x reference)
import jax
import jax.numpy as jnp
from jax.experimental import pallas as pl


def kernel(x, edge_index, edge_attr, batch, W_e1, b_e1, W1, b1, W_e2, b_e2, W2, b2, W_e3, b_e3, W3, b3, gamma, beta):
    raise NotImplementedError("write your pallas kernel here")


# speedup vs baseline: 2.2067x; 2.2067x over previous
import functools

import jax
import jax.numpy as jnp
from jax import lax
from jax.experimental import pallas as pl
from jax.experimental.pallas import tpu as pltpu
from jax.experimental.pallas import tpu_sc as plsc

N = 10000
E = 320000
D = 128
ED = 16
G = 64

NC = 2
NS = 16
NW = NC * NS

CH = 64
IB = 4
NBLK = 40
CPW = NBLK * IB
TOTCH = NW * CPW
EP = TOTCH * CH
NPAD = 8 * 632 + 8 * 624


def _zero_buf(buf):
    @pl.loop(0, CH)
    def _(r):
        for j in range(D // 16):
            buf[r, pl.ds(j * 16, 16)] = jnp.zeros((16,), jnp.float32)


def _compute_msg(xbuf, ebuf, mbuf):
    @plsc.parallel_loop(0, CH)
    def _(r):
        for j in range(D // 16):
            sl = pl.ds(j * 16, 16)
            mbuf[r, sl] = jnp.maximum(xbuf[r, sl] + ebuf[r, sl], 0.0)


def _gine_sc_body(h_hbm, e_hbm, idx_hbm, out_hbm,
                  sidi, e0, e1, x0, x1, m0, m1, agg,
                  ge0, ge1, ee0, ee1, ss0, ss1):
    c = lax.axis_index("c")
    s = lax.axis_index("s")
    w = s * NC + c

    _zero_buf(x0)

    @pl.when(s < 8)
    def _():
        base = s * 632
        for k in range(9):
            pltpu.sync_copy(x0, agg.at[pl.ds(base + k * CH, CH)])
        pltpu.sync_copy(x0.at[pl.ds(0, 56)], agg.at[pl.ds(base + 9 * CH, 56)])

    @pl.when(s >= 8)
    def _():
        base = s * 624 + 64
        for k in range(9):
            pltpu.sync_copy(x0, agg.at[pl.ds(base + k * CH, CH)])
        pltpu.sync_copy(x0.at[pl.ds(0, 48)], agg.at[pl.ds(base + 9 * CH, 48)])

    plsc.subcore_barrier()

    ebufs = (e0, e1)
    xbufs = (x0, x1)
    mbufs = (m0, m1)
    gsems = (ge0, ge1)
    esems = (ee0, ee1)
    ssems = (ss0, ss1)

    def issue(b, r):
        g = w * CPW + b * IB + r
        e, x = ebufs[r % 2], xbufs[r % 2]
        pltpu.async_copy(e_hbm.at[pl.ds(g * CH, CH)], e, esems[r % 2])
        pltpu.async_copy(h_hbm.at[sidi.at[r]], x, gsems[r % 2])

    def wait_data(b, r):
        g = w * CPW + b * IB + r
        e, x = ebufs[r % 2], xbufs[r % 2]
        pltpu.make_async_copy(e_hbm.at[pl.ds(g * CH, CH)], e,
                              esems[r % 2]).wait()
        pltpu.make_async_copy(h_hbm.at[sidi.at[r]], x, gsems[r % 2]).wait()

    def scatter(r):
        m = mbufs[r % 2]
        pltpu.async_copy(m, agg.at[sidi.at[IB + r]], ssems[r % 2], add=True)

    def wait_scatter(r):
        m = mbufs[r % 2]
        pltpu.make_async_copy(m, agg.at[sidi.at[IB + r]], ssems[r % 2]).wait()

    @pl.loop(0, NBLK)
    def _(b):
        @pl.when(b > 0)
        def _():
            wait_scatter(IB - 2)
            wait_scatter(IB - 1)

        pltpu.sync_copy(idx_hbm.at[pl.ds((w * NBLK + b) * 2 * IB, 2 * IB)],
                        sidi)
        issue(b, 0)
        issue(b, 1)
        for r in range(IB):
            wait_data(b, r)
            if r >= 2:
                wait_scatter(r - 2)
            _compute_msg(xbufs[r % 2], ebufs[r % 2], mbufs[r % 2])
            scatter(r)
            if r + 2 < IB:
                issue(b, r + 2)

    wait_scatter(IB - 2)
    wait_scatter(IB - 1)

    plsc.subcore_barrier()

    @pl.when(s < 8)
    def _():
        base = s * 632
        pltpu.sync_copy(agg.at[pl.ds(base, 632)],
                        out_hbm.at[c, pl.ds(base, 632)])

    @pl.when(s >= 8)
    def _():
        base = s * 624 + 64
        pltpu.sync_copy(agg.at[pl.ds(base, 624)],
                        out_hbm.at[c, pl.ds(base, 624)])


@functools.partial(
    pl.kernel,
    out_type=jax.ShapeDtypeStruct((NC, NPAD, D), jnp.float32),
    mesh=plsc.VectorSubcoreMesh(core_axis_name="c", subcore_axis_name="s"),
    scratch_types=[
        pltpu.VMEM((2 * IB, CH), jnp.int32),
        pltpu.VMEM((CH, D), jnp.float32),
        pltpu.VMEM((CH, D), jnp.float32),
        pltpu.VMEM((CH, D), jnp.float32),
        pltpu.VMEM((CH, D), jnp.float32),
        pltpu.VMEM((CH, D), jnp.float32),
        pltpu.VMEM((CH, D), jnp.float32),
        pltpu.VMEM_SHARED((NPAD, D), jnp.float32),
        pltpu.SemaphoreType.DMA,
        pltpu.SemaphoreType.DMA,
        pltpu.SemaphoreType.DMA,
        pltpu.SemaphoreType.DMA,
        pltpu.SemaphoreType.DMA,
        pltpu.SemaphoreType.DMA,
    ],
)
def _gine_sc(h_hbm, e_hbm, idx_hbm, out_hbm, *scratch):
    _gine_sc_body(h_hbm, e_hbm, idx_hbm, out_hbm, *scratch)


_EBLK = 10240


def _eproj_body(ea_ref, w1_ref, b1_ref, w2_ref, b2_ref, w3_ref, b3_ref,
                o1_ref, o2_ref, o3_ref):
    ea = ea_ref[...]
    o1_ref[...] = jnp.dot(ea, w1_ref[...],
                          preferred_element_type=jnp.float32) + b1_ref[...]
    o2_ref[...] = jnp.dot(ea, w2_ref[...],
                          preferred_element_type=jnp.float32) + b2_ref[...]
    o3_ref[...] = jnp.dot(ea, w3_ref[...],
                          preferred_element_type=jnp.float32) + b3_ref[...]


def _eproj(ea_p, wt1, b1, wt2, b2, wt3, b3):
    nblk = EP // _EBLK
    full = lambda shape: pl.BlockSpec(shape, lambda i: (0,) * len(shape))
    return pl.pallas_call(
        _eproj_body,
        grid=(nblk,),
        in_specs=[
            pl.BlockSpec((_EBLK, ED), lambda i: (i, 0)),
            full((ED, D)), full((1, D)),
            full((ED, D)), full((1, D)),
            full((ED, D)), full((1, D)),
        ],
        out_specs=[pl.BlockSpec((_EBLK, D), lambda i: (i, 0))] * 3,
        out_shape=[jax.ShapeDtypeStruct((EP, D), jnp.float32)] * 3,
    )(ea_p, wt1, b1, wt2, b2, wt3, b3)


_NBLK = 2000


def _node_body(h_ref, a_ref, wt_ref, b_ref, o_ref):
    hs = h_ref[...] + a_ref[0] + a_ref[1]
    o_ref[...] = jnp.maximum(
        jnp.dot(hs, wt_ref[...], preferred_element_type=jnp.float32)
        + b_ref[...], 0.0)


def _node_update(h, agg, wt, b):
    nblk = N // _NBLK
    return pl.pallas_call(
        _node_body,
        grid=(nblk,),
        in_specs=[
            pl.BlockSpec((_NBLK, D), lambda i: (i, 0)),
            pl.BlockSpec((NC, _NBLK, D), lambda i: (0, i, 0)),
            pl.BlockSpec((D, D), lambda i: (0, 0)),
            pl.BlockSpec((1, D), lambda i: (0, 0)),
        ],
        out_specs=pl.BlockSpec((_NBLK, D), lambda i: (i, 0)),
        out_shape=jax.ShapeDtypeStruct((N, D), jnp.float32),
    )(h, agg, wt, b)


def _final_body(h_ref, a_ref, batch_ref, w3_ref, b3_ref, gamma_ref, beta_ref,
                o_ref):
    h3 = h_ref[...] + a_ref[0, :N] + a_ref[1, :N]
    gids = lax.broadcasted_iota(jnp.int32, (G, N), 0)
    oh = (batch_ref[...] == gids).astype(jnp.float32)
    counts = jnp.sum(oh, axis=1, keepdims=True)
    hsum = jnp.dot(oh, h3, preferred_element_type=jnp.float32,
                   precision=lax.Precision.HIGHEST)
    s = (jnp.sum(hsum * w3_ref[...], axis=1, keepdims=True)
         + counts * b3_ref[0, 0])
    pooled = s / jnp.maximum(counts, 1.0)
    mean = jnp.mean(pooled)
    var = jnp.mean((pooled - mean) ** 2)
    normed = ((pooled - mean) * lax.rsqrt(var + 1e-5) * gamma_ref[0, 0]
              + beta_ref[0, 0])
    o_ref[...] = jax.nn.sigmoid(normed)


def _final(h2, agg3, batch2d, w3, b3, gamma, beta):
    full = lambda shape: pl.BlockSpec(shape, lambda: (0,) * len(shape))
    return pl.pallas_call(
        _final_body,
        in_specs=[
            full((N, D)), full((NC, NPAD, D)), full((1, N)),
            full((1, D)), full((1, 1)), full((1, 1)), full((1, 1)),
        ],
        out_specs=full((G, 1)),
        out_shape=jax.ShapeDtypeStruct((G, 1), jnp.float32),
    )(h2, agg3, batch2d, w3, b3, gamma, beta)


def kernel(x, edge_index, edge_attr, batch, W_e1, b_e1, W1, b1,
           W_e2, b_e2, W2, b2, W_e3, b_e3, W3, b3, gamma, beta):
    src = edge_index[0]
    dst = edge_index[1]
    padn = EP - E
    src_p = jnp.concatenate(
        [src, jnp.zeros((padn,), jnp.int32)]).reshape(NW, NBLK, 1, IB, CH)
    dst_p = jnp.concatenate(
        [dst, jnp.full((padn,), N, jnp.int32)]).reshape(NW, NBLK, 1, IB, CH)
    idx_p = jnp.concatenate([src_p, dst_p], axis=2).reshape(-1, CH)
    ea_p = jnp.concatenate(
        [edge_attr, jnp.zeros((padn, ED), jnp.float32)])

    e1, e2, e3 = _eproj(ea_p, W_e1.T, b_e1[None], W_e2.T, b_e2[None],
                        W_e3.T, b_e3[None])

    agg1 = _gine_sc(x, e1, idx_p)
    h1 = _node_update(x, agg1, W1.T, b1[None])
    agg2 = _gine_sc(h1, e2, idx_p)
    h2 = _node_update(h1, agg2, W2.T, b2[None])
    agg3 = _gine_sc(h2, e3, idx_p)
    return _final(h2, agg3, batch[None], W3, b3[None], gamma[None], beta[None])
